# baseline (speedup 1.0000x reference)
import jax
import jax.numpy as jnp
from jax.experimental import pallas as pl
from jax.experimental.pallas import tpu as pltpu

_INDEX = 10


def _slice_body(x0_hbm, x1_hbm, o0_hbm, o1_hbm, b0, b1, sem0, sem1, so0, so1):
    in0 = pltpu.make_async_copy(x0_hbm.at[:, _INDEX], b0, sem0)
    in1 = pltpu.make_async_copy(x1_hbm.at[:, _INDEX], b1, sem1)
    in0.start()
    in1.start()
    in0.wait()
    out0 = pltpu.make_async_copy(b0, o0_hbm, so0)
    out0.start()
    in1.wait()
    out1 = pltpu.make_async_copy(b1, o1_hbm, so1)
    out1.start()
    out0.wait()
    out1.wait()


def kernel(x0, x1):
    B, S, D = x0.shape
    any_spec = pl.BlockSpec(memory_space=pltpu.MemorySpace.HBM)
    return pl.pallas_call(
        _slice_body,
        in_specs=[any_spec, any_spec],
        out_specs=[any_spec, any_spec],
        out_shape=[
            jax.ShapeDtypeStruct((B, D), x0.dtype),
            jax.ShapeDtypeStruct((B, D), x1.dtype),
        ],
        scratch_shapes=[
            pltpu.VMEM((B, D), x0.dtype),
            pltpu.VMEM((B, D), x1.dtype),
            pltpu.SemaphoreType.DMA,
            pltpu.SemaphoreType.DMA,
            pltpu.SemaphoreType.DMA,
            pltpu.SemaphoreType.DMA,
        ],
    )(x0, x1)
